# Initial kernel scaffold; baseline (speedup 1.0000x reference)
import jax
import jax.numpy as jnp
from jax.experimental import pallas as pl


def kernel(x_cat, x_cont, tables, W1, b1, bn_w, bn_b):
    raise NotImplementedError("write your pallas kernel here")


# speedup vs baseline: 8.1039x; 8.1039x over previous
import functools

import jax
import jax.numpy as jnp
from jax import lax
from jax.experimental import pallas as pl
from jax.experimental.pallas import tpu as pltpu
from jax.experimental.pallas import tpu_sc as plsc

B = 16384
N_FIELDS = 26
VOCAB = 100000
EMB_DIM = 32
N_CONT = 13
M_LENGTH = 128
N_EMB = N_FIELDS * EMB_DIM

TOT = B * N_FIELDS
NW = 32
PER_W = TOT // NW
CHUNK = 128
NCHUNK = PER_W // CHUNK
SUP = 8
NSUP = NCHUNK // SUP
SUP_ROWS = SUP * CHUNK

_mesh = plsc.VectorSubcoreMesh(core_axis_name="c", subcore_axis_name="s")


@functools.partial(
    pl.kernel,
    mesh=_mesh,
    compiler_params=pltpu.CompilerParams(use_tc_tiling_on_sc=False),
    out_type=jax.ShapeDtypeStruct((TOT, EMB_DIM), jnp.float32),
    scratch_types=[
        pltpu.VMEM((NCHUNK, CHUNK), jnp.int32),
        pltpu.VMEM((2, SUP_ROWS, EMB_DIM), jnp.float32),
        pltpu.SemaphoreType.DMA,
        pltpu.SemaphoreType.DMA,
    ],
)
def _sc_gather(xcat_hbm, tab_hbm, out_hbm, idx_v, rows_v, gsem, ssem):
    cid = lax.axis_index("c")
    sid = lax.axis_index("s")
    wid = sid * 2 + cid
    base = wid * PER_W

    pltpu.sync_copy(xcat_hbm.at[wid], idx_v)

    def _add_offsets(j, carry):
        for k in range(CHUNK // 16):
            pos = base + j * CHUNK + k * 16 + lax.iota(jnp.int32, 16)
            off = lax.rem(pos, N_FIELDS) * VOCAB
            idx_v[j, pl.ds(k * 16, 16)] = idx_v[j, pl.ds(k * 16, 16)] + off
        return carry

    lax.fori_loop(0, NCHUNK, _add_offsets, 0)

    pending = [None, None]
    for s in range(NSUP):
        p = s % 2
        if pending[p] is not None:
            pending[p].wait()
            pending[p] = None
        gathers = []
        for c in range(SUP):
            g = pltpu.async_copy(
                tab_hbm.at[idx_v.at[s * SUP + c]],
                rows_v.at[p, pl.ds(c * CHUNK, CHUNK)],
                gsem,
            )
            gathers.append(g)
        for g in gathers:
            g.wait()
        pending[p] = pltpu.async_copy(
            rows_v.at[p],
            out_hbm.at[pl.ds(base + s * SUP_ROWS, SUP_ROWS)],
            ssem,
        )
    for p in range(2):
        if pending[p] is not None:
            pending[p].wait()


TILE_B = 1024
GRID = B // TILE_B


def _mlp_body(xc_ref, emb_ref, w1e_ref, w1c_ref, b1_ref, bnw_ref, bnb_ref,
              out_ref, stat_scr):
    i = pl.program_id(0)

    @pl.when(i == 0)
    def _():
        xc = xc_ref[...]
        mean = jnp.mean(xc, axis=0)
        var = jnp.mean(xc * xc, axis=0) - mean * mean
        s = bnw_ref[...] * lax.rsqrt(var + 1e-5)
        stat_scr[0, :] = s
        stat_scr[1, :] = bnb_ref[...] - mean * s

    s = stat_scr[0, :]
    t = stat_scr[1, :]
    xcb = xc_ref[pl.ds(i * TILE_B, TILE_B), :]
    x2 = xcb * s[None, :] + t[None, :]
    acc = lax.dot_general(emb_ref[...], w1e_ref[...],
                          (((1,), (1,)), ((), ())),
                          preferred_element_type=jnp.float32)
    acc = acc + lax.dot_general(x2, w1c_ref[...],
                                (((1,), (1,)), ((), ())),
                                preferred_element_type=jnp.float32)
    out_ref[...] = jnp.maximum(acc + b1_ref[...][None, :], 0.0)


_mlp = pl.pallas_call(
    _mlp_body,
    grid=(GRID,),
    in_specs=[
        pl.BlockSpec((B, N_CONT), lambda i: (0, 0)),
        pl.BlockSpec((TILE_B, N_EMB), lambda i: (i, 0)),
        pl.BlockSpec((M_LENGTH, N_EMB), lambda i: (0, 0)),
        pl.BlockSpec((M_LENGTH, N_CONT), lambda i: (0, 0)),
        pl.BlockSpec((M_LENGTH,), lambda i: (0,)),
        pl.BlockSpec((N_CONT,), lambda i: (0,)),
        pl.BlockSpec((N_CONT,), lambda i: (0,)),
    ],
    out_specs=pl.BlockSpec((TILE_B, M_LENGTH), lambda i: (i, 0)),
    out_shape=jax.ShapeDtypeStruct((B, M_LENGTH), jnp.float32),
    scratch_shapes=[pltpu.VMEM((2, N_CONT), jnp.float32)],
)


def kernel(x_cat, x_cont, tables, W1, b1, bn_w, bn_b):
    flat_tab = tables.reshape(N_FIELDS * VOCAB, EMB_DIM)
    xcat_slabs = x_cat.astype(jnp.int32).reshape(NW, NCHUNK, CHUNK)
    emb_flat = _sc_gather(xcat_slabs, flat_tab)
    emb = emb_flat.reshape(B, N_EMB)
    w1e = W1[:, :N_EMB]
    w1c = W1[:, N_EMB:]
    return _mlp(x_cont, emb, w1e, w1c, b1, bn_w, bn_b)
